# baseline (speedup 1.0000x reference)
import functools

import jax
import jax.numpy as jnp
from jax import lax
from jax.experimental import pallas as pl
from jax.experimental.pallas import tpu as pltpu
from jax.experimental.pallas import tpu_sc as plsc

B = 16384
D = 64
EPS_BN = 1e-5
EPS_NORM = 1e-12

NC, NS = 2, 16
NW = NC * NS
BPW = B // NW
CH = 128
NCH = BPW // CH


def _sc_gather_body(user_table, user_idx, item_table, item_idx,
                    u_out, i_out, uidx_v, iidx_v, urows_v, irows_v, sem):
    wid = lax.axis_index("s") * NC + lax.axis_index("c")
    base = wid * BPW
    pltpu.sync_copy(user_idx.at[pl.ds(wid * NCH, NCH)], uidx_v)
    pltpu.sync_copy(item_idx.at[pl.ds(wid * NCH, NCH)], iidx_v)
    copies = []
    for j in range(NCH):
        copies.append(pltpu.async_copy(
            user_table.at[uidx_v.at[j]], urows_v.at[pl.ds(j * CH, CH)], sem))
        copies.append(pltpu.async_copy(
            item_table.at[iidx_v.at[j]], irows_v.at[pl.ds(j * CH, CH)], sem))
    for c in copies:
        c.wait()
    pltpu.sync_copy(urows_v, u_out.at[pl.ds(base, BPW)])
    pltpu.sync_copy(irows_v, i_out.at[pl.ds(base, BPW)])


@functools.cache
def _sc_gather():
    return pl.kernel(
        _sc_gather_body,
        mesh=plsc.VectorSubcoreMesh(core_axis_name="c", subcore_axis_name="s"),
        compiler_params=pltpu.CompilerParams(use_tc_tiling_on_sc=False),
        out_type=[jax.ShapeDtypeStruct((B, D), jnp.float32),
                  jax.ShapeDtypeStruct((B, D), jnp.float32)],
        scratch_types=[pltpu.VMEM((NCH, CH), jnp.int32),
                       pltpu.VMEM((NCH, CH), jnp.int32),
                       pltpu.VMEM((BPW, D), jnp.float32),
                       pltpu.VMEM((BPW, D), jnp.float32),
                       pltpu.SemaphoreType.DMA],
    )


def _towers_body(ue, ie,
                 uW1, ub1, ug1, uc1, uW2, ub2, ug2, uc2, uW3, ub3,
                 iW1, ib1, ig1, ic1, iW2, ib2, ig2, ic2, iW3, ib3,
                 out_ref):
    def bn_relu(y, g, c):
        m = jnp.mean(y, axis=0, keepdims=True)
        d = y - m
        v = jnp.mean(d * d, axis=0, keepdims=True)
        return jnp.maximum(g * d / jnp.sqrt(v + EPS_BN) + c, 0.0)

    def tower(x, W1, b1, g1, c1, W2, b2, g2, c2, W3, b3):
        y = jnp.dot(x, W1[...], preferred_element_type=jnp.float32) + b1[...]
        y = bn_relu(y, g1[...], c1[...])
        y = jnp.dot(y, W2[...], preferred_element_type=jnp.float32) + b2[...]
        y = bn_relu(y, g2[...], c2[...])
        return jnp.dot(y, W3[...], preferred_element_type=jnp.float32) + b3[...]

    def l2norm(x):
        n = jnp.sqrt(jnp.sum(x * x, axis=-1, keepdims=True))
        return x / jnp.maximum(n, EPS_NORM)

    u = l2norm(tower(ue[...], uW1, ub1, ug1, uc1, uW2, ub2, ug2, uc2, uW3, ub3))
    i = l2norm(tower(ie[...], iW1, ib1, ig1, ic1, iW2, ib2, ig2, ic2, iW3, ib3))
    out_ref[...] = jnp.sum(u * i, axis=-1, keepdims=True)


_towers = pl.pallas_call(
    _towers_body,
    out_shape=jax.ShapeDtypeStruct((B, 1), jnp.float32),
)


def _tower_args(tp):
    W, b = tp["W"], tp["b"]
    g, c = tp["gamma"], tp["beta"]
    r = lambda v: v.reshape(1, -1)
    return (W[0], r(b[0]), r(g[0]), r(c[0]),
            W[1], r(b[1]), r(g[1]), r(c[1]),
            W[2], r(b[2]))


def kernel(user_ids, item_ids, params):
    uidx = user_ids.astype(jnp.int32).reshape(NW * NCH, CH)
    iidx = item_ids.astype(jnp.int32).reshape(NW * NCH, CH)
    ue, ie = _sc_gather()(params["user_table"], uidx, params["item_table"], iidx)
    scores = _towers(ue, ie,
                     *_tower_args(params["user_tower"]),
                     *_tower_args(params["item_tower"]))
    return scores.reshape(B)


# speedup vs baseline: 1.5584x; 1.5584x over previous
import functools

import jax
import jax.numpy as jnp
from jax import lax
from jax.experimental import pallas as pl
from jax.experimental.pallas import tpu as pltpu
from jax.experimental.pallas import tpu_sc as plsc

B = 16384
D = 64
EPS_BN = 1e-5
EPS_NORM = 1e-12

NC, NS = 2, 16
NW = NC * NS
BPW = B // NW


def _sc_gather_body(table, idx_hbm, out, idx_v, rows_v, sem):
    wid = lax.axis_index("s") * NC + lax.axis_index("c")
    base = wid * BPW
    pltpu.sync_copy(idx_hbm.at[pl.ds(base, BPW)], idx_v)

    def enqueue(k, _):
        vec = idx_v[pl.ds(k * 16, 16)]
        for j in range(16):
            pltpu.async_copy(table.at[pl.ds(vec[j], 1)],
                             rows_v.at[pl.ds(k * 16 + j, 1)], sem)
        return 0

    lax.fori_loop(0, BPW // 16, enqueue, 0)
    pltpu.make_async_copy(table.at[pl.ds(0, BPW)], rows_v, sem).wait()
    pltpu.sync_copy(rows_v, out.at[pl.ds(base, BPW)])


@functools.cache
def _sc_gather():
    return pl.kernel(
        _sc_gather_body,
        mesh=plsc.VectorSubcoreMesh(core_axis_name="c", subcore_axis_name="s"),
        compiler_params=pltpu.CompilerParams(use_tc_tiling_on_sc=True),
        out_type=jax.ShapeDtypeStruct((B, D), jnp.float32),
        scratch_types=[pltpu.VMEM((BPW,), jnp.int32),
                       pltpu.VMEM((BPW, D), jnp.float32),
                       pltpu.SemaphoreType.DMA],
    )


def _bn_relu(y, g, c):
    m = jnp.mean(y, axis=0, keepdims=True)
    d = y - m
    v = jnp.mean(d * d, axis=0, keepdims=True)
    return jnp.maximum(g * d / jnp.sqrt(v + EPS_BN) + c, 0.0)


def _tower(x, W1, b1, g1, c1, W2, b2, g2, c2, W3, b3):
    y = jnp.dot(x, W1[...], preferred_element_type=jnp.float32) + b1[...]
    y = _bn_relu(y, g1[...], c1[...])
    y = jnp.dot(y, W2[...], preferred_element_type=jnp.float32) + b2[...]
    y = _bn_relu(y, g2[...], c2[...])
    return jnp.dot(y, W3[...], preferred_element_type=jnp.float32) + b3[...]


def _l2norm(x):
    n = jnp.sqrt(jnp.sum(x * x, axis=-1, keepdims=True))
    return x / jnp.maximum(n, EPS_NORM)


def _user_tower_body(ue, W1, b1, g1, c1, W2, b2, g2, c2, W3, b3, out_ref):
    out_ref[...] = _l2norm(_tower(ue[...], W1, b1, g1, c1, W2, b2, g2, c2, W3, b3))


def _item_tower_body(ie, W1, b1, g1, c1, W2, b2, g2, c2, W3, b3, un, out_ref):
    i = _l2norm(_tower(ie[...], W1, b1, g1, c1, W2, b2, g2, c2, W3, b3))
    out_ref[...] = jnp.sum(i * un[...], axis=-1, keepdims=True)


_user_tower = pl.pallas_call(
    _user_tower_body,
    out_shape=jax.ShapeDtypeStruct((B, 32), jnp.float32),
    compiler_params=pltpu.CompilerParams(vmem_limit_bytes=48 * 1024 * 1024),
)

_item_tower = pl.pallas_call(
    _item_tower_body,
    out_shape=jax.ShapeDtypeStruct((B, 1), jnp.float32),
    compiler_params=pltpu.CompilerParams(vmem_limit_bytes=48 * 1024 * 1024),
)


def _tower_args(tp):
    W, b = tp["W"], tp["b"]
    g, c = tp["gamma"], tp["beta"]
    r = lambda v: v.reshape(1, -1)
    return (W[0], r(b[0]), r(g[0]), r(c[0]),
            W[1], r(b[1]), r(g[1]), r(c[1]),
            W[2], r(b[2]))


def kernel(user_ids, item_ids, params):
    gather = _sc_gather()
    ue = gather(params["user_table"], user_ids.astype(jnp.int32))
    ie = gather(params["item_table"], item_ids.astype(jnp.int32))
    un = _user_tower(ue, *_tower_args(params["user_tower"]))
    scores = _item_tower(ie, *_tower_args(params["item_tower"]), un)
    return scores.reshape(B)
